# Initial kernel scaffold; baseline (speedup 1.0000x reference)
import functools

import jax
import jax.numpy as jnp
from jax import lax
from jax.experimental import pallas as pl
from jax.experimental.pallas import tpu as pltpu
from jax.experimental.pallas import tpu_sc as plsc

_K = 256
_NC = 2
_NS = 16
_NW = _NC * _NS
_L = 16
_UNROLL = 8


def _orderable(v):
    b = plsc.bitcast(v, jnp.int32)
    o = b ^ (lax.shift_right_arithmetic(b, 31) & jnp.int32(0x7FFFFFFF))
    return jnp.where(o == jnp.int32(-1), jnp.int32(0), o)


def _count_cmp(ord_ref, cand, nv, strict):
    cand_v = jnp.full((_L,), cand, jnp.int32)

    def cnt(i, acc):
        for j in range(_UNROLL):
            o = ord_ref[pl.ds((i * _UNROLL + j) * _L, _L)]
            m = (o > cand_v) if strict else (o >= cand_v)
            acc = acc + m.astype(jnp.int32)
        return acc

    acc = lax.fori_loop(0, nv // _UNROLL, cnt, jnp.zeros((_L,), jnp.int32))
    return jnp.sum(acc)


def _row_topk_mask(ord_ref, out_ref, r, nv):
    def bitstep(bi, t):
        cand = t + lax.shift_left(jnp.int32(1), jnp.int32(31) - bi)
        c = _count_cmp(ord_ref, cand, nv, strict=False)
        return jnp.where(c >= _K, cand, t)

    t0 = jnp.int32(-2147483647 - 1)
    thr = lax.fori_loop(0, 32, bitstep, t0)

    c_gt = _count_cmp(ord_ref, thr, nv, strict=True)
    need = jnp.int32(_K) - c_gt
    thr_v = jnp.full((_L,), thr, jnp.int32)

    def mask_step(i, off):
        o = ord_ref[pl.ds(i * _L, _L)]
        m_gt = o > thr_v
        m_eq = o == thr_v
        e = m_eq.astype(jnp.int32)
        pc = plsc.cumsum(e)
        rank = (pc - e) + off
        sel = m_gt | (m_eq & (rank < need))
        out_ref[r, pl.ds(i * _L, _L)] = jnp.where(sel, 1.0, 0.0).astype(jnp.float32)
        return off + jnp.sum(e)

    lax.fori_loop(0, nv, mask_step, jnp.int32(0))


@functools.cache
def _build(bsz, slen):
    rows_per = bsz // _NW
    nv = slen // _L
    mesh = plsc.VectorSubcoreMesh(
        core_axis_name="c", subcore_axis_name="s",
        num_cores=_NC, num_subcores=_NS,
    )

    @functools.partial(
        pl.kernel,
        out_type=jax.ShapeDtypeStruct((bsz, slen), jnp.float32),
        mesh=mesh,
        scratch_types=[
            pltpu.VMEM((rows_per, slen), jnp.float32),
            pltpu.VMEM((rows_per, slen), jnp.float32),
            pltpu.VMEM((slen,), jnp.int32),
        ],
    )
    def k(score_hbm, out_hbm, rows_v, out_v, ord_v):
        wid = lax.axis_index("s") * _NC + lax.axis_index("c")
        base = wid * rows_per
        pltpu.sync_copy(score_hbm.at[pl.ds(base, rows_per)], rows_v)
        for r in range(rows_per):
            def pre(i, _):
                for j in range(_UNROLL):
                    idx = (i * _UNROLL + j) * _L
                    v = rows_v[r, pl.ds(idx, _L)]
                    ord_v[pl.ds(idx, _L)] = _orderable(v)
                return 0

            lax.fori_loop(0, nv // _UNROLL, pre, 0)
            _row_topk_mask(ord_v, out_v, r, nv)
        pltpu.sync_copy(out_v, out_hbm.at[pl.ds(base, rows_per)])

    return k


@jax.jit
def kernel(score):
    bsz, slen = score.shape
    return _build(bsz, slen)(score)


# speedup vs baseline: 5.1854x; 5.1854x over previous
import functools

import jax
import jax.numpy as jnp
from jax import lax
from jax.experimental import pallas as pl
from jax.experimental.pallas import tpu as pltpu
from jax.experimental.pallas import tpu_sc as plsc

_K = 256
_NC = 2
_NS = 16
_NW = _NC * _NS
_L = 16
_UNROLL = 8


def _orderable(v):
    b = lax.bitcast_convert_type(v, jnp.int32)
    return b ^ (lax.shift_right_arithmetic(b, 31) & jnp.int32(0x7FFFFFFF))


def _count_cmp(ord_ref, cand, nv, strict):
    cand_v = jnp.full((_L,), cand, jnp.int32)

    def cnt(i, acc):
        for j in range(_UNROLL):
            o = ord_ref[pl.ds((i * _UNROLL + j) * _L, _L)]
            m = (o > cand_v) if strict else (o >= cand_v)
            acc = acc + m.astype(jnp.int32)
        return acc

    acc = lax.fori_loop(0, nv // _UNROLL, cnt, jnp.zeros((_L,), jnp.int32))
    return jnp.sum(acc)


def _row_topk_mask(ord_ref, out_ref, r, nv):
    def bitstep(bi, t):
        cand = t + lax.shift_left(jnp.int32(1), jnp.int32(31) - bi)
        c = _count_cmp(ord_ref, cand, nv, strict=False)
        return jnp.where(c >= _K, cand, t)

    t0 = jnp.int32(-2147483647 - 1)
    thr = lax.fori_loop(0, 32, bitstep, t0)

    c_gt = _count_cmp(ord_ref, thr, nv, strict=True)
    need = jnp.int32(_K) - c_gt
    thr_v = jnp.full((_L,), thr, jnp.int32)

    def mask_step(i, off):
        o = ord_ref[pl.ds(i * _L, _L)]
        m_gt = o > thr_v
        m_eq = o == thr_v
        e = m_eq.astype(jnp.int32)
        pc = plsc.cumsum(e)
        rank = (pc - e) + off
        sel = m_gt | (m_eq & (rank < need))
        out_ref[r, pl.ds(i * _L, _L)] = jnp.where(sel, 1.0, 0.0).astype(jnp.float32)
        return off + jnp.sum(e)

    lax.fori_loop(0, nv, mask_step, jnp.int32(0))


@functools.cache
def _build(bsz, slen):
    rows_per = bsz // _NW
    nv = slen // _L
    mesh = plsc.VectorSubcoreMesh(
        core_axis_name="c", subcore_axis_name="s",
        num_cores=_NC, num_subcores=_NS,
    )

    @functools.partial(
        pl.kernel,
        out_type=jax.ShapeDtypeStruct((bsz, slen), jnp.float32),
        mesh=mesh,
        scratch_types=[
            pltpu.VMEM((rows_per, slen), jnp.float32),
            pltpu.VMEM((rows_per, slen), jnp.float32),
            pltpu.VMEM((slen,), jnp.int32),
        ],
        compiler_params=pltpu.CompilerParams(needs_layout_passes=False),
    )
    def k(score_hbm, out_hbm, rows_v, out_v, ord_v):
        wid = lax.axis_index("s") * _NC + lax.axis_index("c")
        base = wid * rows_per
        pltpu.sync_copy(score_hbm.at[pl.ds(base, rows_per)], rows_v)
        for r in range(rows_per):
            def pre(i, _):
                for j in range(_UNROLL):
                    idx = (i * _UNROLL + j) * _L
                    v = rows_v[r, pl.ds(idx, _L)]
                    ord_v[pl.ds(idx, _L)] = _orderable(v)
                return 0

            lax.fori_loop(0, nv // _UNROLL, pre, 0)
            _row_topk_mask(ord_v, out_v, r, nv)
        pltpu.sync_copy(out_v, out_hbm.at[pl.ds(base, rows_per)])

    return k


@jax.jit
def kernel(score):
    bsz, slen = score.shape
    return _build(bsz, slen)(score)
